# Initial kernel scaffold; baseline (speedup 1.0000x reference)
import jax
import jax.numpy as jnp
from jax.experimental import pallas as pl


def kernel(x, Wg, W1, W2):
    raise NotImplementedError("write your pallas kernel here")


# speedup vs baseline: 1.0646x; 1.0646x over previous
import functools

import jax
import jax.numpy as jnp
from jax.experimental import pallas as pl
from jax.experimental.pallas import tpu as pltpu

HIDDEN = 1024
FF = 2816
E = 8
TOKENS = 2048
TILE_M = 256
NT = TOKENS // TILE_M


def _gate_body(x_ref, wg_ref, c_ref):
    x = x_ref[...]
    wg = wg_ref[...]
    logits = jax.lax.dot_general(
        x, wg, (((1,), (1,)), ((), ())),
        preferred_element_type=jnp.float32,
        precision=jax.lax.Precision.DEFAULT,
    )
    lane = jax.lax.broadcasted_iota(jnp.int32, logits.shape, 1)
    big = jnp.float32(-1e30)
    m0 = jnp.max(logits, axis=1, keepdims=True)
    i0 = jnp.min(jnp.where(logits == m0, lane, E), axis=1, keepdims=True)
    l2 = jnp.where(lane == i0, big, logits)
    m1 = jnp.max(l2, axis=1, keepdims=True)
    i1 = jnp.min(jnp.where(l2 == m1, lane, E), axis=1, keepdims=True)
    e1 = jnp.exp(m1 - m0)
    s0 = 1.0 / (1.0 + e1)
    s1 = e1 / (1.0 + e1)
    c_ref[...] = jnp.where(lane == i0, s0, 0.0) + jnp.where(lane == i1, s1, 0.0)


def _ffn_body(x_ref, w1_ref, w2_ref, c_ref, o_ref, acc_ref):
    e = pl.program_id(0)
    t = pl.program_id(1)
    xb = x_ref[...].astype(jnp.bfloat16)
    w1 = w1_ref[0]
    w2 = w2_ref[0]
    z = jax.lax.dot_general(xb, w1, (((1,), (1,)), ((), ())),
                            preferred_element_type=jnp.float32)
    h = z * jax.nn.sigmoid(z)
    y = jax.lax.dot_general(h.astype(jnp.bfloat16), w2, (((1,), (1,)), ((), ())),
                            preferred_element_type=jnp.float32)
    lane = jax.lax.broadcasted_iota(jnp.int32, (TILE_M, E), 1)
    w = jnp.sum(c_ref[...] * (lane == e).astype(jnp.float32), axis=1,
                keepdims=True)
    contrib = y * w
    rows = pl.ds(t * TILE_M, TILE_M)

    @pl.when(e == 0)
    def _():
        acc_ref[rows, :] = contrib

    @pl.when(e != 0)
    def _():
        acc_ref[rows, :] = acc_ref[rows, :] + contrib

    @pl.when(e == E - 1)
    def _():
        o_ref[...] = acc_ref[rows, :]


@jax.jit
def kernel(x, Wg, W1, W2):
    b, t, d = x.shape
    h = x.reshape(t, d)

    combine = pl.pallas_call(
        _gate_body,
        out_shape=jax.ShapeDtypeStruct((TOKENS, E), jnp.float32),
    )(h, Wg)

    w1b = W1.astype(jnp.bfloat16)
    w2b = W2.astype(jnp.bfloat16)

    y = pl.pallas_call(
        _ffn_body,
        grid=(E, NT),
        in_specs=[
            pl.BlockSpec((TILE_M, HIDDEN), lambda e, i: (i, 0)),
            pl.BlockSpec((1, FF, HIDDEN), lambda e, i: (e, 0, 0)),
            pl.BlockSpec((1, HIDDEN, FF), lambda e, i: (e, 0, 0)),
            pl.BlockSpec((TILE_M, E), lambda e, i: (i, 0)),
        ],
        out_specs=pl.BlockSpec((TILE_M, HIDDEN), lambda e, i: (i, 0)),
        out_shape=jax.ShapeDtypeStruct((TOKENS, HIDDEN), jnp.float32),
        scratch_shapes=[pltpu.VMEM((TOKENS, HIDDEN), jnp.float32)],
    )(h, w1b, w2b, combine)

    return y.reshape(b, t, d)
